# baseline (speedup 1.0000x reference)
import functools
import math

import jax
import jax.numpy as jnp
from jax import lax
from jax.experimental import pallas as pl
from jax.experimental.pallas import tpu as pltpu
from jax.experimental.pallas import tpu_sc as plsc

B, N, K, D, A = 16, 2048, 16, 128, 10
ROWS = B * N
NW = 32
RPW = ROWS // NW
CH = 8
IDXW = 128
IPC = CH * K
NB = 4
NCH = RPW // CH
NIO = NCH // NB
IROWS = RPW * K // IDXW


@functools.cache
def _make_gather_max():
    mesh = plsc.VectorSubcoreMesh(core_axis_name="c", subcore_axis_name="s")

    @functools.partial(
        pl.kernel,
        out_type=jax.ShapeDtypeStruct((ROWS, D), jnp.float32),
        mesh=mesh,
        scratch_types=[
            pltpu.VMEM((IROWS, IDXW), jnp.int32),
            pltpu.VMEM((NB, IPC, D), jnp.float32),
            pltpu.VMEM((NB, CH, D), jnp.float32),
            pltpu.SemaphoreType.DMA,
            pltpu.SemaphoreType.DMA,
            pltpu.SemaphoreType.DMA,
            pltpu.SemaphoreType.DMA,
            pltpu.SemaphoreType.DMA,
            pltpu.SemaphoreType.DMA,
            pltpu.SemaphoreType.DMA,
            pltpu.SemaphoreType.DMA,
        ],
    )
    def _gather_max(h_hbm, idx_hbm, out_hbm, idx_v, rows_v, out_v,
                    sem_g0, sem_g1, sem_g2, sem_g3,
                    sem_s0, sem_s1, sem_s2, sem_s3):
        wid = lax.axis_index("s") * 2 + lax.axis_index("c")
        row0 = wid * RPW
        bbase = (row0 // N) * N
        sem_g = (sem_g0, sem_g1, sem_g2, sem_g3)
        sem_s = (sem_s0, sem_s1, sem_s2, sem_s3)

        pltpu.sync_copy(idx_hbm.at[wid], idx_v)

        @plsc.parallel_loop(0, IROWS, 1, unroll=4)
        def rebase(j):
            for jj in range(IDXW // 16):
                sl = pl.ds(jj * 16, 16)
                idx_v[j, sl] = idx_v[j, sl] + bbase

        def gather_copy(c, bb):
            return pltpu.make_async_copy(
                h_hbm.at[idx_v.at[c]], rows_v.at[bb], sem_g[bb])

        def compute(c, bb):
            @plsc.parallel_loop(0, CH, 1, unroll=4)
            def node(ci):
                base = ci * K
                for j in range(D // 16):
                    sl = pl.ds(j * 16, 16)
                    acc = rows_v[bb, base, sl]
                    for k in range(1, K):
                        acc = jnp.maximum(acc, rows_v[bb, base + k, sl])
                    out_v[bb, ci, sl] = acc

        def store_copy(c, bb):
            return pltpu.make_async_copy(
                out_v.at[bb], out_hbm.at[pl.ds(row0 + c * CH, CH)], sem_s[bb])

        for c in range(NB - 1):
            gather_copy(c, c).start()

        def body(io, carry):
            for u in range(NB):
                c = NB * io + u
                gather_copy(c, u).wait()

                @pl.when(c + NB - 1 < NCH)
                def _():
                    gather_copy(c + NB - 1, (u + NB - 1) % NB).start()

                @pl.when(io > 0)
                def _():
                    store_copy(c - NB, u).wait()

                compute(c, u)
                store_copy(c, u).start()
            return carry

        lax.fori_loop(0, NIO, body, 0)
        for u in range(NB):
            store_copy(NCH - NB + u, u).wait()

    return _gather_max


def _encode_body(x_ref, w_ref, b_ref, o_ref):
    o_ref[...] = jnp.maximum(
        jnp.dot(x_ref[...], w_ref[...], preferred_element_type=jnp.float32)
        + b_ref[...],
        0.0,
    )


def _mlp_body(h_ref, p_ref, w_ref, b_ref, o_ref):
    cat = jnp.concatenate([h_ref[...], p_ref[...]], axis=-1)
    acc = jnp.dot(cat, w_ref[...], preferred_element_type=jnp.float32)
    o_ref[...] = jnp.maximum(acc + b_ref[...], 0.0)


def _decode_body(h1_ref, p_ref, w2_ref, b2_ref, wg_ref, bg_ref, qt_ref,
                 lo_ref, s_ref):
    cat2 = jnp.concatenate([h1_ref[...], p_ref[...]], axis=-1)
    h = jnp.dot(cat2, w2_ref[...], preferred_element_type=jnp.float32)
    h = jnp.maximum(h + b2_ref[...], 0.0)
    g = jnp.mean(h, axis=0, keepdims=True)
    cat = jnp.concatenate([h, jnp.broadcast_to(g, h.shape)], axis=-1)
    hg = jnp.dot(cat, wg_ref[...], preferred_element_type=jnp.float32)
    hg = jnp.maximum(hg + bg_ref[...], 0.0)
    logits = jnp.dot(hg, qt_ref[...], preferred_element_type=jnp.float32)
    logits = logits / jnp.sqrt(jnp.float32(D))
    m = jnp.max(logits, axis=1, keepdims=True)
    shifted = logits - m
    lsm = shifted - jnp.log(
        jnp.sum(jnp.exp(shifted), axis=1, keepdims=True))
    lo_ref[...] = lsm
    mm = jnp.max(lsm, axis=1, keepdims=True)
    io = lax.broadcasted_iota(jnp.int32, (lsm.shape[0], A), 1)
    s_ref[...] = jnp.min(jnp.where(lsm == mm, io, A), axis=1, keepdims=True)


def _full(shape):
    return pl.BlockSpec(shape, lambda i: (0, 0))


def kernel(step, inputs, nn_idx, W1, b1, W2, b2, Wg, bg, Q):
    x = inputs.reshape(ROWS, 2)
    idx_slab = nn_idx.astype(jnp.int32).reshape(NW, IROWS, IDXW)
    b1r = b1.reshape(1, D)
    b2r = b2.reshape(1, D)
    bgr = bg.reshape(1, D)
    QT = Q.T

    h = pl.pallas_call(
        _encode_body,
        grid=(NW,),
        in_specs=[
            pl.BlockSpec((RPW, 2), lambda i: (i, 0)),
            _full((2, D)),
            _full((1, D)),
        ],
        out_specs=pl.BlockSpec((RPW, D), lambda i: (i, 0)),
        out_shape=jax.ShapeDtypeStruct((ROWS, D), jnp.float32),
    )(x, W1, b1r)

    gather_max = _make_gather_max()
    pooled = gather_max(h, idx_slab)
    h = pl.pallas_call(
        _mlp_body,
        grid=(NW,),
        in_specs=[
            pl.BlockSpec((RPW, D), lambda i: (i, 0)),
            pl.BlockSpec((RPW, D), lambda i: (i, 0)),
            _full((2 * D, D)),
            _full((1, D)),
        ],
        out_specs=pl.BlockSpec((RPW, D), lambda i: (i, 0)),
        out_shape=jax.ShapeDtypeStruct((ROWS, D), jnp.float32),
    )(h, pooled, W2, b2r)

    pooled = gather_max(h, idx_slab)
    logits, samples = pl.pallas_call(
        _decode_body,
        grid=(B,),
        in_specs=[
            pl.BlockSpec((N, D), lambda i: (i, 0)),
            pl.BlockSpec((N, D), lambda i: (i, 0)),
            _full((2 * D, D)),
            _full((1, D)),
            _full((2 * D, D)),
            _full((1, D)),
            _full((D, A)),
        ],
        out_specs=[
            pl.BlockSpec((N, A), lambda i: (i, 0)),
            pl.BlockSpec((N, 1), lambda i: (i, 0)),
        ],
        out_shape=[
            jax.ShapeDtypeStruct((ROWS, A), jnp.float32),
            jax.ShapeDtypeStruct((ROWS, 1), jnp.int32),
        ],
    )(h, pooled, W2, b2r, Wg, bgr, QT)

    return logits.reshape(B, N, A), samples.reshape(B, N)[:, None, :]


# speedup vs baseline: 1.1864x; 1.1864x over previous
import functools

import jax
import jax.numpy as jnp
from jax import lax
from jax.experimental import pallas as pl
from jax.experimental.pallas import tpu as pltpu
from jax.experimental.pallas import tpu_sc as plsc

B, N, K, D, A = 16, 2048, 16, 128, 10
ROWS = B * N
HB = B // 2
HROWS = ROWS // 2
NW = 32
RPW = HROWS // NW
CH = 16
IDXW = 128
IPC = CH * K
NCH = RPW // CH
NIO = NCH // 2
IROWS = RPW * K // IDXW
MGRID = HROWS // 1024


@functools.cache
def _make_gather_max():
    mesh = plsc.VectorSubcoreMesh(core_axis_name="c", subcore_axis_name="s")

    @functools.partial(
        pl.kernel,
        out_type=jax.ShapeDtypeStruct((HROWS, D), jnp.float32),
        mesh=mesh,
        scratch_types=[
            pltpu.VMEM((IROWS, IDXW), jnp.int32),
            pltpu.VMEM((2, IPC, D), jnp.float32),
            pltpu.VMEM((2, CH, D), jnp.float32),
            pltpu.SemaphoreType.DMA,
            pltpu.SemaphoreType.DMA,
            pltpu.SemaphoreType.DMA,
            pltpu.SemaphoreType.DMA,
        ],
    )
    def _gather_max(h_hbm, idx_hbm, out_hbm, idx_v, rows_v, out_v,
                    sem_g0, sem_g1, sem_s0, sem_s1):
        wid = lax.axis_index("s") * 2 + lax.axis_index("c")
        row0 = wid * RPW
        bbase = (row0 // N) * N
        sem_g = (sem_g0, sem_g1)
        sem_s = (sem_s0, sem_s1)

        pltpu.sync_copy(idx_hbm.at[wid], idx_v)

        @plsc.parallel_loop(0, IROWS, 1, unroll=4)
        def rebase(j):
            for jj in range(IDXW // 16):
                sl = pl.ds(jj * 16, 16)
                idx_v[j, sl] = idx_v[j, sl] + bbase

        def start_gather(c, bb):
            for half in range(IPC // IDXW):
                pltpu.async_copy(
                    h_hbm.at[idx_v.at[2 * c + half]],
                    rows_v.at[bb, pl.ds(half * IDXW, IDXW)],
                    sem_g[bb],
                )

        def wait_gather(c, bb):
            for half in range(IPC // IDXW):
                pltpu.make_async_copy(
                    h_hbm.at[idx_v.at[2 * c + half]],
                    rows_v.at[bb, pl.ds(half * IDXW, IDXW)],
                    sem_g[bb],
                ).wait()

        def compute(c, bb):
            @plsc.parallel_loop(0, CH, 1, unroll=4)
            def node(ci):
                base = ci * K
                for j in range(D // 16):
                    sl = pl.ds(j * 16, 16)
                    acc = rows_v[bb, base, sl]
                    for k in range(1, K):
                        acc = jnp.maximum(acc, rows_v[bb, base + k, sl])
                    out_v[bb, ci, sl] = acc

        def store_copy(c, bb):
            return pltpu.make_async_copy(
                out_v.at[bb], out_hbm.at[pl.ds(row0 + c * CH, CH)], sem_s[bb])

        start_gather(0, 0)

        def body(io, carry):
            c0 = 2 * io
            c1 = c0 + 1
            start_gather(c1, 1)
            wait_gather(c0, 0)

            @pl.when(io > 0)
            def _():
                store_copy(c0 - 2, 0).wait()

            compute(c0, 0)
            store_copy(c0, 0).start()

            @pl.when(io + 1 < NIO)
            def _():
                start_gather(c0 + 2, 0)

            wait_gather(c1, 1)

            @pl.when(io > 0)
            def _():
                store_copy(c1 - 2, 1).wait()

            compute(c1, 1)
            store_copy(c1, 1).start()
            return carry

        lax.fori_loop(0, NIO, body, 0)
        store_copy(NCH - 2, 0).wait()
        store_copy(NCH - 1, 1).wait()

    return _gather_max


def _encode_body(x_ref, w_ref, b_ref, o_ref):
    o_ref[...] = jnp.maximum(
        jnp.dot(x_ref[...], w_ref[...], preferred_element_type=jnp.float32)
        + b_ref[...],
        0.0,
    )


def _mlp_body(h_ref, p_ref, w_ref, b_ref, o_ref):
    cat = jnp.concatenate([h_ref[...], p_ref[...]], axis=-1)
    acc = jnp.dot(cat, w_ref[...], preferred_element_type=jnp.float32)
    o_ref[...] = jnp.maximum(acc + b_ref[...], 0.0)


def _decode_body(h1_ref, p_ref, w2_ref, b2_ref, wg_ref, bg_ref, qt_ref,
                 lo_ref, s_ref):
    cat2 = jnp.concatenate([h1_ref[...], p_ref[...]], axis=-1)
    h = jnp.dot(cat2, w2_ref[...], preferred_element_type=jnp.float32)
    h = jnp.maximum(h + b2_ref[...], 0.0)
    g = jnp.mean(h, axis=0, keepdims=True)
    cat = jnp.concatenate([h, jnp.broadcast_to(g, h.shape)], axis=-1)
    hg = jnp.dot(cat, wg_ref[...], preferred_element_type=jnp.float32)
    hg = jnp.maximum(hg + bg_ref[...], 0.0)
    logits = jnp.dot(hg, qt_ref[...], preferred_element_type=jnp.float32)
    logits = logits / jnp.sqrt(jnp.float32(D))
    m = jnp.max(logits, axis=1, keepdims=True)
    shifted = logits - m
    lsm = shifted - jnp.log(
        jnp.sum(jnp.exp(shifted), axis=1, keepdims=True))
    lo_ref[...] = lsm
    mm = jnp.max(lsm, axis=1, keepdims=True)
    io = lax.broadcasted_iota(jnp.int32, (lsm.shape[0], A), 1)
    s_ref[...] = jnp.min(jnp.where(lsm == mm, io, A), axis=1, keepdims=True)


def _full(shape):
    return pl.BlockSpec(shape, lambda i: (0, 0))


def _encode(xh, W1, b1r):
    return pl.pallas_call(
        _encode_body,
        grid=(MGRID,),
        in_specs=[
            pl.BlockSpec((1024, 2), lambda i: (i, 0)),
            _full((2, D)),
            _full((1, D)),
        ],
        out_specs=pl.BlockSpec((1024, D), lambda i: (i, 0)),
        out_shape=jax.ShapeDtypeStruct((HROWS, D), jnp.float32),
    )(xh, W1, b1r)


def _mlp(h, pooled, W2, b2r):
    return pl.pallas_call(
        _mlp_body,
        grid=(MGRID,),
        in_specs=[
            pl.BlockSpec((1024, D), lambda i: (i, 0)),
            pl.BlockSpec((1024, D), lambda i: (i, 0)),
            _full((2 * D, D)),
            _full((1, D)),
        ],
        out_specs=pl.BlockSpec((1024, D), lambda i: (i, 0)),
        out_shape=jax.ShapeDtypeStruct((HROWS, D), jnp.float32),
    )(h, pooled, W2, b2r)


def _decode(h, pooled, W2, b2r, Wg, bgr, QT):
    return pl.pallas_call(
        _decode_body,
        grid=(HB,),
        in_specs=[
            pl.BlockSpec((N, D), lambda i: (i, 0)),
            pl.BlockSpec((N, D), lambda i: (i, 0)),
            _full((2 * D, D)),
            _full((1, D)),
            _full((2 * D, D)),
            _full((1, D)),
            _full((D, A)),
        ],
        out_specs=[
            pl.BlockSpec((N, A), lambda i: (i, 0)),
            pl.BlockSpec((N, 1), lambda i: (i, 0)),
        ],
        out_shape=[
            jax.ShapeDtypeStruct((HROWS, A), jnp.float32),
            jax.ShapeDtypeStruct((HROWS, 1), jnp.int32),
        ],
    )(h, pooled, W2, b2r, Wg, bgr, QT)


def kernel(step, inputs, nn_idx, W1, b1, W2, b2, Wg, bg, Q):
    x = inputs.reshape(2, HROWS, 2)
    idx2 = nn_idx.astype(jnp.int32).reshape(2, NW, IROWS, IDXW)
    b1r = b1.reshape(1, D)
    b2r = b2.reshape(1, D)
    bgr = bg.reshape(1, D)
    QT = Q.T

    gm = _make_gather_max()

    h0a = _encode(x[0], W1, b1r)
    h0b = _encode(x[1], W1, b1r)
    p1a = gm(h0a, idx2[0])
    p1b = gm(h0b, idx2[1])
    h1a = _mlp(h0a, p1a, W2, b2r)
    p2a = gm(h1a, idx2[0])
    h1b = _mlp(h0b, p1b, W2, b2r)
    p2b = gm(h1b, idx2[1])
    loa, sa = _decode(h1a, p2a, W2, b2r, Wg, bgr, QT)
    lob, sb = _decode(h1b, p2b, W2, b2r, Wg, bgr, QT)

    logits = jnp.concatenate([loa, lob], axis=0).reshape(B, N, A)
    samples = jnp.concatenate([sa, sb], axis=0).reshape(B, N)[:, None, :]
    return logits, samples
